# Initial kernel scaffold; baseline (speedup 1.0000x reference)
import jax
import jax.numpy as jnp
from jax.experimental import pallas as pl


def kernel(word_ids, table):
    raise NotImplementedError("write your pallas kernel here")


# speedup vs baseline: 7.4861x; 7.4861x over previous
import jax
import jax.numpy as jnp
from jax import lax
from jax.experimental import pallas as pl
from jax.experimental.pallas import tpu as pltpu
from jax.experimental.pallas import tpu_sc as plsc

B, T, L, D, V = 4096, 20, 20, 128, 30000
K = T * L
NC, NS = 2, 16
NW = NC * NS
BPW = B // NW
CH = 80
NCH = K // CH
ROWS_PW = BPW * NCH


def _encoder_body(idx_hbm, tab_hbm, out_hbm, idx_v, rows_v, out_v, sem):
    wid = lax.axis_index("s") * NC + lax.axis_index("c")
    pltpu.sync_copy(idx_hbm.at[pl.ds(wid * ROWS_PW, ROWS_PW)], idx_v)

    def batch_body(b, carry):
        def chunk_body(c, accs):
            r = b * NCH + c
            pltpu.async_copy(tab_hbm.at[idx_v.at[r]], rows_v, sem).wait()

            def row_body(j, a):
                return tuple(
                    a[k] + rows_v[j, pl.ds(16 * k, 16)] for k in range(8)
                )

            return lax.fori_loop(0, CH, row_body, accs)

        accs = tuple(jnp.zeros((16,), jnp.float32) for _ in range(8))
        accs = lax.fori_loop(0, NCH, chunk_body, accs)
        scale = jnp.float32(1.0 / K)
        for k in range(8):
            out_v[b, pl.ds(16 * k, 16)] = accs[k] * scale
        return carry

    lax.fori_loop(0, BPW, batch_body, 0)
    pltpu.sync_copy(out_v, out_hbm.at[pl.ds(wid * BPW, BPW)])


def kernel(word_ids, table):
    idx = word_ids.reshape(NW * ROWS_PW, CH)
    mesh = plsc.VectorSubcoreMesh(core_axis_name="c", subcore_axis_name="s")
    f = pl.kernel(
        _encoder_body,
        mesh=mesh,
        out_type=jax.ShapeDtypeStruct((B, D), jnp.float32),
        scratch_types=[
            pltpu.VMEM((ROWS_PW, CH), jnp.int32),
            pltpu.VMEM((CH, D), jnp.float32),
            pltpu.VMEM((BPW, D), jnp.float32),
            pltpu.SemaphoreType.DMA,
        ],
    )
    return f(idx, table)


# speedup vs baseline: 9.6107x; 1.2838x over previous
import jax
import jax.numpy as jnp
from jax import lax
from jax.experimental import pallas as pl
from jax.experimental.pallas import tpu as pltpu
from jax.experimental.pallas import tpu_sc as plsc

B, T, L, D, V = 4096, 20, 20, 128, 30000
K = T * L
NC, NS = 2, 16
NW = NC * NS
BPW = B // NW
CH = 80
NCH = K // CH
ROWS_PW = BPW * NCH
TOTAL = ROWS_PW
PAIR = 2 * NCH


def _encoder_body(idx_hbm, tab_hbm, out_hbm, idx_v, rows0, rows1, out_v,
                  sem0, sem1):
    wid = lax.axis_index("s") * NC + lax.axis_index("c")
    pltpu.sync_copy(idx_hbm.at[pl.ds(wid * ROWS_PW, ROWS_PW)], idx_v)
    bufs = (rows0, rows1)
    sems = (sem0, sem1)
    zero8 = tuple(jnp.zeros((16,), jnp.float32) for _ in range(8))
    scale = jnp.float32(1.0 / K)

    def start(g, p):
        pltpu.async_copy(tab_hbm.at[idx_v.at[g]], bufs[p], sems[p])

    def wait(g, p):
        pltpu.make_async_copy(tab_hbm.at[idx_v.at[g]], bufs[p], sems[p]).wait()

    def reduce_chunk(buf, accs):
        def rb(j, a):
            mid = tuple(a[k] + buf[2 * j, pl.ds(16 * k, 16)] for k in range(8))
            return tuple(
                mid[k] + buf[2 * j + 1, pl.ds(16 * k, 16)] for k in range(8)
            )

        return lax.fori_loop(0, CH // 2, rb, accs)

    start(0, 0)

    def pair_body(p, carry):
        accs = zero8
        for q in range(PAIR):
            g = p * PAIR + q
            wait(g, q % 2)

            @pl.when(g + 1 < TOTAL)
            def _():
                start(g + 1, (q + 1) % 2)

            accs = reduce_chunk(bufs[q % 2], accs)
            if q % NCH == NCH - 1:
                b = 2 * p + q // NCH
                for k in range(8):
                    out_v[b, pl.ds(16 * k, 16)] = accs[k] * scale
                accs = zero8
        return carry

    lax.fori_loop(0, BPW // 2, pair_body, 0)
    pltpu.sync_copy(out_v, out_hbm.at[pl.ds(wid * BPW, BPW)])


def kernel(word_ids, table):
    idx = word_ids.reshape(NW * ROWS_PW, CH)
    mesh = plsc.VectorSubcoreMesh(core_axis_name="c", subcore_axis_name="s")
    f = pl.kernel(
        _encoder_body,
        mesh=mesh,
        out_type=jax.ShapeDtypeStruct((B, D), jnp.float32),
        scratch_types=[
            pltpu.VMEM((ROWS_PW, CH), jnp.int32),
            pltpu.VMEM((CH, D), jnp.float32),
            pltpu.VMEM((CH, D), jnp.float32),
            pltpu.VMEM((BPW, D), jnp.float32),
            pltpu.SemaphoreType.DMA,
            pltpu.SemaphoreType.DMA,
        ],
    )
    return f(idx, table)


# speedup vs baseline: 10.0380x; 1.0445x over previous
import jax
import jax.numpy as jnp
from jax import lax
from jax.experimental import pallas as pl
from jax.experimental.pallas import tpu as pltpu
from jax.experimental.pallas import tpu_sc as plsc

B, T, L, D, V = 4096, 20, 20, 128, 30000
K = T * L
NC, NS = 2, 16
NW = NC * NS
BPW = B // NW
CH = 80
NCH = K // CH
ROWS_PW = BPW * NCH
TOTAL = ROWS_PW
PAIR = 2 * NCH
RU = 4


def _encoder_body(idx_hbm, tab_hbm, out_hbm, idx_v, rows0, rows1, out_v,
                  sem0, sem1):
    wid = lax.axis_index("s") * NC + lax.axis_index("c")
    pltpu.sync_copy(idx_hbm.at[pl.ds(wid * ROWS_PW, ROWS_PW)], idx_v)
    bufs = (rows0, rows1)
    sems = (sem0, sem1)
    zero16 = tuple(jnp.zeros((16,), jnp.float32) for _ in range(16))
    scale = jnp.float32(1.0 / K)
    lane = lax.iota(jnp.int32, 16)
    cols_a = tuple(32 * c + 2 * lane for c in range(4))
    cols_b = tuple(32 * c + 1 + 2 * lane for c in range(4))

    def start(g, p):
        pltpu.async_copy(tab_hbm.at[idx_v.at[g]], bufs[p], sems[p])

    def wait(g, p):
        pltpu.make_async_copy(tab_hbm.at[idx_v.at[g]], bufs[p], sems[p]).wait()

    himask = jnp.full((16,), jnp.int32(-65536))

    def reduce_chunk(buf, accs):
        def rb(j, a):
            a = list(a)
            for u in range(RU):
                for c in range(4):
                    x = buf[RU * j + u, pl.ds(16 * c, 16)]
                    ea = plsc.bitcast(x << 16, jnp.float32)
                    eb = plsc.bitcast(x & himask, jnp.float32)
                    a[2 * c] = a[2 * c] + ea
                    a[2 * c + 1] = a[2 * c + 1] + eb
            return tuple(a)

        return lax.fori_loop(0, CH // RU, rb, accs)

    start(0, 0)

    def pair_body(p, carry):
        accs = zero16
        for q in range(PAIR):
            g = p * PAIR + q
            wait(g, q % 2)

            @pl.when(g + 1 < TOTAL)
            def _():
                start(g + 1, (q + 1) % 2)

            accs = reduce_chunk(bufs[q % 2], accs)
            if q % NCH == NCH - 1:
                b = 2 * p + q // NCH
                row = jnp.full((16,), b, jnp.int32)
                for c in range(4):
                    plsc.store_scatter(
                        out_v, [row, cols_a[c]], accs[2 * c] * scale)
                    plsc.store_scatter(
                        out_v, [row, cols_b[c]], accs[2 * c + 1] * scale)
                accs = zero16
        return carry

    lax.fori_loop(0, BPW // 2, pair_body, 0)
    pltpu.sync_copy(out_v, out_hbm.at[pl.ds(wid * BPW, BPW)])


def kernel(word_ids, table):
    idx = word_ids.reshape(NW * ROWS_PW, CH)
    tab16 = table.astype(jnp.bfloat16)
    tab_pk = lax.bitcast_convert_type(
        tab16.reshape(V, D // 2, 2), jnp.int32)
    mesh = plsc.VectorSubcoreMesh(core_axis_name="c", subcore_axis_name="s")
    f = pl.kernel(
        _encoder_body,
        mesh=mesh,
        compiler_params=pltpu.CompilerParams(
            needs_layout_passes=False, use_tc_tiling_on_sc=False),
        out_type=jax.ShapeDtypeStruct((B, D), jnp.float32),
        scratch_types=[
            pltpu.VMEM((ROWS_PW, CH), jnp.int32),
            pltpu.VMEM((CH, D // 2), jnp.int32),
            pltpu.VMEM((CH, D // 2), jnp.int32),
            pltpu.VMEM((BPW, D), jnp.float32),
            pltpu.SemaphoreType.DMA,
            pltpu.SemaphoreType.DMA,
        ],
    )
    return f(idx, tab_pk)


# speedup vs baseline: 17.7839x; 1.7717x over previous
import jax
import jax.numpy as jnp
from jax import lax
from jax.experimental import pallas as pl
from jax.experimental.pallas import tpu as pltpu
from jax.experimental.pallas import tpu_sc as plsc

B, T, L, D, V = 4096, 20, 20, 128, 30000
K = T * L
NC, NS = 2, 16
NW = NC * NS
BPW = B // NW
CH = 80
NCH = K // CH
ROWS_PW = BPW * NCH
TOTAL = ROWS_PW
NBUF = 4
UU = 4
PAIR = UU * NCH
RU = 4


def _encoder_body(idx_hbm, tab_hbm, out_hbm, idx_v, rows0, rows1, rows2,
                  rows3, out_v, sem0, sem1, sem2, sem3):
    wid = lax.axis_index("s") * NC + lax.axis_index("c")
    pltpu.sync_copy(idx_hbm.at[pl.ds(wid * ROWS_PW, ROWS_PW)], idx_v)
    bufs = (rows0, rows1, rows2, rows3)
    sems = (sem0, sem1, sem2, sem3)
    zero16 = tuple(jnp.zeros((16,), jnp.float32) for _ in range(16))
    scale = jnp.float32(1.0 / K)
    lane = lax.iota(jnp.int32, 16)
    cols_a = tuple(32 * c + 2 * lane for c in range(4))
    cols_b = tuple(32 * c + 1 + 2 * lane for c in range(4))

    def start(g, p):
        pltpu.async_copy(tab_hbm.at[idx_v.at[g]], bufs[p], sems[p])

    def wait(g, p):
        pltpu.make_async_copy(tab_hbm.at[idx_v.at[g]], bufs[p], sems[p]).wait()

    def reduce_chunk(buf, accs):
        def rb(j, a):
            a = list(a)
            for u in range(RU):
                for c in range(4):
                    x = buf[RU * j + u, pl.ds(16 * c, 16)]
                    ea = plsc.bitcast(x << 16, jnp.float32)
                    eb = plsc.bitcast(x, jnp.float32)
                    a[2 * c] = a[2 * c] + ea
                    a[2 * c + 1] = a[2 * c + 1] + eb
            return tuple(a)

        return lax.fori_loop(0, CH // RU, rb, accs)

    for g0 in range(NBUF - 1):
        start(g0, g0)

    def step_body(p, carry):
        accs = zero16
        for q in range(PAIR):
            g = p * PAIR + q
            wait(g, q % NBUF)

            @pl.when(g + NBUF - 1 < TOTAL)
            def _():
                start(g + NBUF - 1, (q + NBUF - 1) % NBUF)

            accs = reduce_chunk(bufs[q % NBUF], accs)
            if q % NCH == NCH - 1:
                b = UU * p + q // NCH
                row = jnp.full((16,), b, jnp.int32)
                for c in range(4):
                    plsc.store_scatter(
                        out_v, [row, cols_a[c]], accs[2 * c] * scale)
                    plsc.store_scatter(
                        out_v, [row, cols_b[c]], accs[2 * c + 1] * scale)
                accs = zero16
        return carry

    lax.fori_loop(0, BPW // UU, step_body, 0)
    pltpu.sync_copy(out_v, out_hbm.at[pl.ds(wid * BPW, BPW)])


def kernel(word_ids, table):
    idx = word_ids.reshape(NW * ROWS_PW, CH)
    tab16 = table.astype(jnp.bfloat16)
    tab_pk = lax.bitcast_convert_type(
        tab16.reshape(V, D // 2, 2), jnp.int32)
    mesh = plsc.VectorSubcoreMesh(core_axis_name="c", subcore_axis_name="s")
    f = pl.kernel(
        _encoder_body,
        mesh=mesh,
        compiler_params=pltpu.CompilerParams(
            needs_layout_passes=False, use_tc_tiling_on_sc=False),
        out_type=jax.ShapeDtypeStruct((B, D), jnp.float32),
        scratch_types=[
            pltpu.VMEM((ROWS_PW, CH), jnp.int32),
            pltpu.VMEM((CH, D // 2), jnp.int32),
            pltpu.VMEM((CH, D // 2), jnp.int32),
            pltpu.VMEM((CH, D // 2), jnp.int32),
            pltpu.VMEM((CH, D // 2), jnp.int32),
            pltpu.VMEM((BPW, D), jnp.float32),
            pltpu.SemaphoreType.DMA,
            pltpu.SemaphoreType.DMA,
            pltpu.SemaphoreType.DMA,
            pltpu.SemaphoreType.DMA,
        ],
    )
    return f(idx, tab_pk)


# speedup vs baseline: 26.1647x; 1.4713x over previous
import jax
import jax.numpy as jnp
from jax import lax
from jax.experimental import pallas as pl
from jax.experimental.pallas import tpu as pltpu
from jax.experimental.pallas import tpu_sc as plsc

B, T, L, D, V = 4096, 20, 20, 128, 30000
K = T * L
NC, NS = 2, 16
NW = NC * NS
BPW = B // NW
CH = 80
NCH = K // CH
ROWS_PW = BPW * NCH
TOTAL = ROWS_PW
NBUF = 4
UU = 4
PAIR = UU * NCH
RU = 4


def _encoder_body(idx_hbm, tab_hbm, out_hbm, idx_v, rows0, rows1, rows2,
                  rows3, out_v, sem0, sem1, sem2, sem3):
    wid = lax.axis_index("s") * NC + lax.axis_index("c")
    pltpu.sync_copy(idx_hbm.at[pl.ds(wid * ROWS_PW, ROWS_PW)], idx_v)
    bufs = (rows0, rows1, rows2, rows3)
    sems = (sem0, sem1, sem2, sem3)
    zero16 = tuple(jnp.zeros((16,), jnp.float32) for _ in range(16))
    scale = jnp.float32(1.0 / K)

    def start(g, p):
        pltpu.async_copy(tab_hbm.at[idx_v.at[g]], bufs[p], sems[p])

    def wait(g, p):
        pltpu.make_async_copy(tab_hbm.at[idx_v.at[g]], bufs[p], sems[p]).wait()

    def reduce_chunk(buf, accs):
        def rb(j, a):
            a = list(a)
            for u in range(RU):
                for c in range(4):
                    x = buf[RU * j + u, pl.ds(16 * c, 16)]
                    lo = plsc.bitcast(x << 16, jnp.float32)
                    hi = plsc.bitcast(x, jnp.float32)
                    a[c] = a[c] + lo
                    a[4 + c] = a[4 + c] + hi
            return tuple(a)

        return lax.fori_loop(0, CH // RU, rb, accs)

    for g0 in range(NBUF - 1):
        start(g0, g0)

    def step_body(p, carry):
        accs = zero16
        for q in range(PAIR):
            g = p * PAIR + q
            wait(g, q % NBUF)

            @pl.when(g + NBUF - 1 < TOTAL)
            def _():
                start(g + NBUF - 1, (q + NBUF - 1) % NBUF)

            accs = reduce_chunk(bufs[q % NBUF], accs)
            if q % NCH == NCH - 1:
                b = UU * p + q // NCH
                for c in range(4):
                    out_v[b, pl.ds(16 * c, 16)] = accs[c] * scale
                    out_v[b, pl.ds(64 + 16 * c, 16)] = accs[4 + c] * scale
                accs = zero16
        return carry

    lax.fori_loop(0, BPW // UU, step_body, 0)
    pltpu.sync_copy(out_v, out_hbm.at[pl.ds(wid * BPW, BPW)])


def kernel(word_ids, table):
    idx = word_ids.reshape(NW * ROWS_PW, CH)
    bu = lax.bitcast_convert_type(table, jnp.uint32)
    half = jnp.uint32(0x8000)
    pk = (((bu[:, : D // 2] + half) >> 16)
          | ((bu[:, D // 2:] + half) & jnp.uint32(0xFFFF0000)))
    tab_pk = lax.bitcast_convert_type(pk, jnp.int32)
    mesh = plsc.VectorSubcoreMesh(core_axis_name="c", subcore_axis_name="s")
    f = pl.kernel(
        _encoder_body,
        mesh=mesh,
        compiler_params=pltpu.CompilerParams(
            needs_layout_passes=False, use_tc_tiling_on_sc=False),
        out_type=jax.ShapeDtypeStruct((B, D), jnp.float32),
        scratch_types=[
            pltpu.VMEM((ROWS_PW, CH), jnp.int32),
            pltpu.VMEM((CH, D // 2), jnp.int32),
            pltpu.VMEM((CH, D // 2), jnp.int32),
            pltpu.VMEM((CH, D // 2), jnp.int32),
            pltpu.VMEM((CH, D // 2), jnp.int32),
            pltpu.VMEM((BPW, D), jnp.float32),
            pltpu.SemaphoreType.DMA,
            pltpu.SemaphoreType.DMA,
            pltpu.SemaphoreType.DMA,
            pltpu.SemaphoreType.DMA,
        ],
    )
    return f(idx, tab_pk)
